# baseline (speedup 1.0000x reference)
import jax
import jax.numpy as jnp
from jax.experimental import pallas as pl


_M, _K, _N = 16384, 16384, 64
_BM, _BK = 512, 4096


def _mm_body(w_ref, x_ref, o_ref):
    k = pl.program_id(1)

    @pl.when(k == 0)
    def _():
        o_ref[...] = jnp.zeros_like(o_ref)

    o_ref[...] += jnp.dot(w_ref[...], x_ref[...],
                          preferred_element_type=jnp.float32)


def kernel(x, W):
    grid = (_M // _BM, _K // _BK)
    return pl.pallas_call(
        _mm_body,
        grid=grid,
        in_specs=[
            pl.BlockSpec((_BM, _BK), lambda i, k: (i, k)),
            pl.BlockSpec((_BK, _N), lambda i, k: (k, 0)),
        ],
        out_specs=pl.BlockSpec((_BM, _N), lambda i, k: (i, 0)),
        out_shape=jax.ShapeDtypeStruct((_M, _N), jnp.float32),
    )(W, x)
